# baseline (speedup 1.0000x reference)
import jax
import jax.numpy as jnp
from jax import lax
from jax.experimental import pallas as pl
from jax.experimental.pallas import tpu as pltpu

B = 4
W = 8
N = 16
TW = 4096 * 1024 // (W * N)
NSQ = N * N
K = NSQ // 2
NNZ = NSQ - K - N
EPS = 1e-8
RING = 4


def _copy_corr_kernel2(x_any, x0_ref, xn_any, csum_ref, sem):
    b = pl.program_id(0)
    w = pl.program_id(1)
    i = b * W + w
    slot = lax.rem(i, RING)
    src = x_any.at[b, w]
    dst = xn_any.at[b, :, pl.ds(w * TW, TW)]

    @pl.when(i >= RING)
    def _():
        pltpu.make_async_copy(src, dst, sem.at[slot]).wait()

    pltpu.make_async_copy(src, dst, sem.at[slot]).start()

    @pl.when(b == 0)
    def _():
        x = x0_ref[0, 0]
        mean = jnp.mean(x, axis=1, keepdims=True)
        xc = x - mean
        cov = jax.lax.dot_general(
            xc, xc, (((1,), (1,)), ((), ())),
            preferred_element_type=jnp.float32,
        ) / (TW - 1 + EPS)
        rows = jax.lax.broadcasted_iota(jnp.int32, (N, N), 0)
        cols = jax.lax.broadcasted_iota(jnp.int32, (N, N), 1)
        eye = rows == cols
        var = jnp.sum(jnp.where(eye, cov, 0.0), axis=1, keepdims=True)
        std = jnp.sqrt(var + EPS)
        corr = jnp.clip(cov / (std * std.T + EPS), -1.0, 1.0)

        @pl.when(w == 0)
        def _():
            csum_ref[...] = corr

        @pl.when(w > 0)
        def _():
            csum_ref[...] += corr

    @pl.when(i == B * W - 1)
    def _():
        for r in range(RING):
            pltpu.make_async_copy(src, dst, sem.at[r]).wait()


def _edge_kernel(c_row_ref, c_col_ref, rows_ref, cols_ref):
    c_row = c_row_ref[...]
    c_col = c_col_ref[...]
    less = (c_col < c_row).astype(jnp.float32)
    rank = jnp.sum(less, axis=0, keepdims=True)

    fj = jax.lax.broadcasted_iota(jnp.int32, (1, NSQ), 1)
    r_j = fj // N
    c_j = fj % N
    keep = jnp.logical_and(rank >= K, r_j != c_j).astype(jnp.float32)

    ii = jax.lax.broadcasted_iota(jnp.int32, (NSQ, NSQ), 0)
    jj = jax.lax.broadcasted_iota(jnp.int32, (NSQ, NSQ), 1)
    upper = (ii < jj).astype(jnp.float32)
    pos = jax.lax.dot_general(
        keep, upper, (((1,), (0,)), ((), ())),
        preferred_element_type=jnp.float32,
    )

    slot = jax.lax.broadcasted_iota(jnp.int32, (NSQ, 1), 0).astype(jnp.float32)
    sel = (pos == slot).astype(jnp.float32) * keep
    fi = jax.lax.broadcasted_iota(jnp.int32, (NSQ, 1), 0)
    r_col = (fi // N).astype(jnp.float32)
    c_col_idx = (fi % N).astype(jnp.float32)
    rows_out = jax.lax.dot_general(
        sel, r_col, (((1,), (0,)), ((), ())),
        preferred_element_type=jnp.float32,
    )
    cols_out = jax.lax.dot_general(
        sel, c_col_idx, (((1,), (0,)), ((), ())),
        preferred_element_type=jnp.float32,
    )
    rows_ref[...] = rows_out.astype(jnp.int32)
    cols_ref[...] = cols_out.astype(jnp.int32)


def kernel(H):
    X = H.reshape(B, W, N, TW)
    x_nodes, csum = pl.pallas_call(
        _copy_corr_kernel2,
        grid=(B, W),
        in_specs=[
            pl.BlockSpec(memory_space=pltpu.MemorySpace.HBM),
            pl.BlockSpec(
                (1, 1, N, TW),
                lambda b, w: (0, jnp.where(b == 0, w, W - 1), 0, 0),
            ),
        ],
        out_specs=[
            pl.BlockSpec(memory_space=pltpu.MemorySpace.HBM),
            pl.BlockSpec((N, N), lambda b, w: (0, 0)),
        ],
        out_shape=[
            jax.ShapeDtypeStruct((B, N, W * TW), jnp.float32),
            jax.ShapeDtypeStruct((N, N), jnp.float32),
        ],
        scratch_shapes=[pltpu.SemaphoreType.DMA((RING,))],
        compiler_params=pltpu.CompilerParams(
            dimension_semantics=("arbitrary", "arbitrary"),
        ),
    )(X, X)

    c_row = csum.reshape(1, NSQ)
    c_col = csum.reshape(NSQ, 1)
    rows, cols = pl.pallas_call(
        _edge_kernel,
        in_specs=[
            pl.BlockSpec((1, NSQ), lambda: (0, 0)),
            pl.BlockSpec((NSQ, 1), lambda: (0, 0)),
        ],
        out_specs=[
            pl.BlockSpec((NSQ, 1), lambda: (0, 0)),
            pl.BlockSpec((NSQ, 1), lambda: (0, 0)),
        ],
        out_shape=[
            jax.ShapeDtypeStruct((NSQ, 1), jnp.int32),
            jax.ShapeDtypeStruct((NSQ, 1), jnp.int32),
        ],
    )(c_row, c_col)

    edge_index = jnp.stack([rows[:NNZ, 0], cols[:NNZ, 0]], axis=0)
    return (x_nodes, edge_index)


# speedup vs baseline: 13.9042x; 13.9042x over previous
import functools
import jax
import jax.numpy as jnp
from jax import lax
from jax.experimental import pallas as pl
from jax.experimental.pallas import tpu as pltpu
from jax.experimental.pallas import tpu_sc as plsc

B = 4
W = 8
N = 16
TW = 4096 * 1024 // (W * N)
NSQ = N * N
K = NSQ // 2
NNZ = NSQ - K - N
EPS = 1e-8
RING = 4

L = 16
NSUB = 16
NPAD = 128


def _copy_corr_kernel(x_ref, xn_ref, csum_ref):
    b = pl.program_id(0)
    w = pl.program_id(1)
    x = x_ref[0, 0]
    xn_ref[0] = x

    @pl.when(b == 0)
    def _():
        mean = jnp.mean(x, axis=1, keepdims=True)
        xc = x - mean
        cov = jax.lax.dot_general(
            xc, xc, (((1,), (1,)), ((), ())),
            preferred_element_type=jnp.float32,
        ) / (TW - 1 + EPS)
        rows = jax.lax.broadcasted_iota(jnp.int32, (N, N), 0)
        cols = jax.lax.broadcasted_iota(jnp.int32, (N, N), 1)
        eye = rows == cols
        var = jnp.sum(jnp.where(eye, cov, 0.0), axis=1, keepdims=True)
        std = jnp.sqrt(var + EPS)
        corr = jnp.clip(cov / (std * std.T + EPS), -1.0, 1.0)

        @pl.when(w == 0)
        def _():
            csum_ref[...] = corr

        @pl.when(w > 0)
        def _():
            csum_ref[...] += corr


def _sc_edge_kernel(c_hbm, rows_hbm, cols_hbm,
                    vals_v, mine_v, counts_v, pa_v, myrows_v, mycols_v,
                    mg_v, out_v, counts_sh, rows_sh, cols_sh):
    c = lax.axis_index("c")
    s = lax.axis_index("s")
    lane = lax.iota(jnp.int32, L)
    zero_i = jnp.zeros((L,), jnp.int32)

    pltpu.sync_copy(c_hbm, vals_v)
    pltpu.sync_copy(c_hbm.at[pl.ds(s * L, L)], mine_v)
    mine = mine_v[...]

    rank = zero_i
    for t in range(NSUB):
        vj = vals_v[pl.ds(t * L, L)]
        for m in range(L):
            e = jnp.broadcast_to(
                jnp.sum(jnp.where(lane == m, vj, 0.0)), (L,))
            rank = rank + (e < mine).astype(jnp.int32)
    keep = jnp.logical_and(rank >= K, lane != s)
    keep_i = keep.astype(jnp.int32)
    counts_v[...] = jnp.broadcast_to(jnp.sum(keep_i), (L,))

    @pl.when(c == 0)
    def _():
        pltpu.sync_copy(counts_v, counts_sh.at[pl.ds(s * L, L)])

    plsc.subcore_barrier()

    @pl.when(c == 0)
    def _():
        pltpu.sync_copy(counts_sh, pa_v)
        prefix = zero_i
        for t in range(NSUB):
            prefix = jnp.where(t < s, prefix + pa_v[pl.ds(t * L, L)], prefix)

        pos = prefix + plsc.cumsum(keep_i) - keep_i
        write = jnp.logical_and(keep, pos < NNZ)
        idx = jnp.where(write, pos, NPAD - 1)
        for k in range(NPAD // L):
            myrows_v[pl.ds(k * L, L)] = zero_i
            mycols_v[pl.ds(k * L, L)] = zero_i
        plsc.store_scatter(myrows_v, [idx],
                           jnp.broadcast_to(s, (L,)), mask=write)
        plsc.store_scatter(mycols_v, [idx], lane, mask=write)
        pltpu.sync_copy(myrows_v, rows_sh.at[pl.ds(s * NPAD, NPAD)])
        pltpu.sync_copy(mycols_v, cols_sh.at[pl.ds(s * NPAD, NPAD)])

    plsc.subcore_barrier()

    @pl.when(jnp.logical_and(c == 0, s == 0))
    def _():
        for src_sh, dst_hbm in ((rows_sh, rows_hbm), (cols_sh, cols_hbm)):
            pltpu.sync_copy(src_sh, mg_v)
            acc = [zero_i] * (NPAD // L)
            for t in range(NSUB):
                for k in range(NPAD // L):
                    acc[k] = acc[k] + mg_v[pl.ds(t * NPAD + k * L, L)]
            for k in range(NNZ // L):
                out_v[pl.ds(k * L, L)] = acc[k]
            pltpu.sync_copy(out_v, dst_hbm)


def _sc_edges(csum_flat):
    mesh = plsc.VectorSubcoreMesh(
        core_axis_name="c", subcore_axis_name="s",
        num_cores=2, num_subcores=NSUB)
    run = functools.partial(
        pl.kernel,
        out_type=[jax.ShapeDtypeStruct((NNZ,), jnp.int32),
                  jax.ShapeDtypeStruct((NNZ,), jnp.int32)],
        mesh=mesh,
        scratch_types=[
            pltpu.VMEM((NSQ,), jnp.float32),
            pltpu.VMEM((L,), jnp.float32),
            pltpu.VMEM((L,), jnp.int32),
            pltpu.VMEM((NSUB * L,), jnp.int32),
            pltpu.VMEM((NPAD,), jnp.int32),
            pltpu.VMEM((NPAD,), jnp.int32),
            pltpu.VMEM((NSUB * NPAD,), jnp.int32),
            pltpu.VMEM((NNZ,), jnp.int32),
            pltpu.VMEM_SHARED((NSUB * L,), jnp.int32),
            pltpu.VMEM_SHARED((NSUB * NPAD,), jnp.int32),
            pltpu.VMEM_SHARED((NSUB * NPAD,), jnp.int32),
        ],
        compiler_params=pltpu.CompilerParams(needs_layout_passes=False),
    )(_sc_edge_kernel)
    return run(csum_flat)


def kernel(H):
    X = H.reshape(B, W, N, TW)
    x_nodes, csum = pl.pallas_call(
        _copy_corr_kernel,
        grid=(B, W),
        in_specs=[
            pl.BlockSpec((1, 1, N, TW), lambda b, w: (b, w, 0, 0)),
        ],
        out_specs=[
            pl.BlockSpec((1, N, TW), lambda b, w: (b, 0, w)),
            pl.BlockSpec((N, N), lambda b, w: (0, 0)),
        ],
        out_shape=[
            jax.ShapeDtypeStruct((B, N, W * TW), jnp.float32),
            jax.ShapeDtypeStruct((N, N), jnp.float32),
        ],
        compiler_params=pltpu.CompilerParams(
            dimension_semantics=("arbitrary", "arbitrary"),
        ),
    )(X)

    rows, cols = _sc_edges(csum.reshape(NSQ))
    edge_index = jnp.stack([rows, cols], axis=0)
    return (x_nodes, edge_index)


# speedup vs baseline: 15.5729x; 1.1200x over previous
import jax
import jax.numpy as jnp
from jax.experimental import pallas as pl
from jax.experimental.pallas import tpu as pltpu

B = 4
W = 8
N = 16
TW = 4096 * 1024 // (W * N)
NSQ = N * N
K = NSQ // 2
NNZ = NSQ - K - N
EPS = 1e-8


def _copy_corr_kernel(x_ref, xn_ref, csum_ref):
    b = pl.program_id(0)
    w = pl.program_id(1)
    x = x_ref[0, 0]
    xn_ref[0] = x

    @pl.when(b == 0)
    def _():
        mean = jnp.mean(x, axis=1, keepdims=True)
        xc = x - mean
        cov = jax.lax.dot_general(
            xc, xc, (((1,), (1,)), ((), ())),
            preferred_element_type=jnp.float32,
        ) / (TW - 1 + EPS)
        rows = jax.lax.broadcasted_iota(jnp.int32, (N, N), 0)
        cols = jax.lax.broadcasted_iota(jnp.int32, (N, N), 1)
        eye = rows == cols
        var = jnp.sum(jnp.where(eye, cov, 0.0), axis=1, keepdims=True)
        std = jnp.sqrt(var + EPS)
        corr = jnp.clip(cov / (std * std.T + EPS), -1.0, 1.0)

        @pl.when(w == 0)
        def _():
            csum_ref[...] = corr

        @pl.when(w > 0)
        def _():
            csum_ref[...] += corr


def _edge_kernel(c_row_ref, c_col_ref, rows_ref, cols_ref):
    c_row = c_row_ref[...]
    c_col = c_col_ref[...]
    less = (c_col < c_row).astype(jnp.float32)
    rank = jnp.sum(less, axis=0, keepdims=True)

    fj = jax.lax.broadcasted_iota(jnp.int32, (1, NSQ), 1)
    r_j = fj // N
    c_j = fj % N
    keep = jnp.logical_and(rank >= K, r_j != c_j).astype(jnp.float32)

    ii = jax.lax.broadcasted_iota(jnp.int32, (NSQ, NSQ), 0)
    jj = jax.lax.broadcasted_iota(jnp.int32, (NSQ, NSQ), 1)
    upper = (ii < jj).astype(jnp.float32)
    pos = jax.lax.dot_general(
        keep, upper, (((1,), (0,)), ((), ())),
        preferred_element_type=jnp.float32,
    )

    slot = jax.lax.broadcasted_iota(jnp.int32, (NSQ, 1), 0).astype(jnp.float32)
    sel = (pos == slot).astype(jnp.float32) * keep
    fi = jax.lax.broadcasted_iota(jnp.int32, (NSQ, 1), 0)
    r_col = (fi // N).astype(jnp.float32)
    c_col_idx = (fi % N).astype(jnp.float32)
    rows_out = jax.lax.dot_general(
        sel, r_col, (((1,), (0,)), ((), ())),
        preferred_element_type=jnp.float32,
    )
    cols_out = jax.lax.dot_general(
        sel, c_col_idx, (((1,), (0,)), ((), ())),
        preferred_element_type=jnp.float32,
    )
    rows_ref[...] = rows_out.astype(jnp.int32)
    cols_ref[...] = cols_out.astype(jnp.int32)


def kernel(H):
    X = H.reshape(B, W, N, TW)
    x_nodes, csum = pl.pallas_call(
        _copy_corr_kernel,
        grid=(B, W),
        in_specs=[
            pl.BlockSpec((1, 1, N, TW), lambda b, w: (b, w, 0, 0)),
        ],
        out_specs=[
            pl.BlockSpec((1, N, TW), lambda b, w: (b, 0, w)),
            pl.BlockSpec((N, N), lambda b, w: (0, 0)),
        ],
        out_shape=[
            jax.ShapeDtypeStruct((B, N, W * TW), jnp.float32),
            jax.ShapeDtypeStruct((N, N), jnp.float32),
        ],
        compiler_params=pltpu.CompilerParams(
            dimension_semantics=("arbitrary", "arbitrary"),
        ),
    )(X)

    c_row = csum.reshape(1, NSQ)
    c_col = csum.reshape(NSQ, 1)
    rows, cols = pl.pallas_call(
        _edge_kernel,
        in_specs=[
            pl.BlockSpec((1, NSQ), lambda: (0, 0)),
            pl.BlockSpec((NSQ, 1), lambda: (0, 0)),
        ],
        out_specs=[
            pl.BlockSpec((NSQ, 1), lambda: (0, 0)),
            pl.BlockSpec((NSQ, 1), lambda: (0, 0)),
        ],
        out_shape=[
            jax.ShapeDtypeStruct((NSQ, 1), jnp.int32),
            jax.ShapeDtypeStruct((NSQ, 1), jnp.int32),
        ],
    )(c_row, c_col)

    edge_index = jnp.stack([rows[:NNZ, 0], cols[:NNZ, 0]], axis=0)
    return (x_nodes, edge_index)


# speedup vs baseline: 15.8957x; 1.0207x over previous
import jax
import jax.numpy as jnp
from jax import lax
from jax.experimental import pallas as pl
from jax.experimental.pallas import tpu as pltpu

B = 4
W = 8
N = 16
TW = 4096 * 1024 // (W * N)
NSQ = N * N
K = NSQ // 2
NNZ = NSQ - K - N
EPS = 1e-8
NSTEPS = B * W
RING = 3


def _copy_corr_kernel3(x_any, xn_any, csum_ref, buf, sem_in, sem_out):
    i = pl.program_id(0)
    b = i // W
    w = lax.rem(i, W)
    slot = lax.rem(i, RING)
    nslot = lax.rem(i + 1, RING)

    @pl.when(i == 0)
    def _():
        pltpu.make_async_copy(x_any.at[0, 0], buf.at[0], sem_in.at[0]).start()

    @pl.when(i >= 2)
    def _():
        pltpu.make_async_copy(
            buf.at[nslot], xn_any.at[0, :, pl.ds(0, TW)], sem_out.at[nslot]
        ).wait()

    @pl.when(i + 1 < NSTEPS)
    def _():
        b1 = (i + 1) // W
        w1 = lax.rem(i + 1, W)
        pltpu.make_async_copy(
            x_any.at[b1, w1], buf.at[nslot], sem_in.at[nslot]).start()

    pltpu.make_async_copy(
        x_any.at[b, w], buf.at[slot], sem_in.at[slot]).wait()
    pltpu.make_async_copy(
        buf.at[slot], xn_any.at[b, :, pl.ds(w * TW, TW)], sem_out.at[slot]
    ).start()

    @pl.when(b == 0)
    def _():
        x = buf[slot]
        mean = jnp.mean(x, axis=1, keepdims=True)
        xc = x - mean
        cov = jax.lax.dot_general(
            xc, xc, (((1,), (1,)), ((), ())),
            preferred_element_type=jnp.float32,
        ) / (TW - 1 + EPS)
        rows = jax.lax.broadcasted_iota(jnp.int32, (N, N), 0)
        cols = jax.lax.broadcasted_iota(jnp.int32, (N, N), 1)
        eye = rows == cols
        var = jnp.sum(jnp.where(eye, cov, 0.0), axis=1, keepdims=True)
        std = jnp.sqrt(var + EPS)
        corr = jnp.clip(cov / (std * std.T + EPS), -1.0, 1.0)

        @pl.when(w == 0)
        def _():
            csum_ref[...] = corr

        @pl.when(w > 0)
        def _():
            csum_ref[...] += corr

    @pl.when(i == NSTEPS - 1)
    def _():
        for k in (NSTEPS - 2, NSTEPS - 1):
            pltpu.make_async_copy(
                buf.at[k % RING], xn_any.at[0, :, pl.ds(0, TW)],
                sem_out.at[k % RING],
            ).wait()


def _edge_kernel(c_row_ref, c_col_ref, rows_ref, cols_ref):
    c_row = c_row_ref[...]
    c_col = c_col_ref[...]
    less = (c_col < c_row).astype(jnp.float32)
    rank = jnp.sum(less, axis=0, keepdims=True)

    fj = jax.lax.broadcasted_iota(jnp.int32, (1, NSQ), 1)
    r_j = fj // N
    c_j = fj % N
    keep = jnp.logical_and(rank >= K, r_j != c_j).astype(jnp.float32)

    ii = jax.lax.broadcasted_iota(jnp.int32, (NSQ, NSQ), 0)
    jj = jax.lax.broadcasted_iota(jnp.int32, (NSQ, NSQ), 1)
    upper = (ii < jj).astype(jnp.float32)
    pos = jax.lax.dot_general(
        keep, upper, (((1,), (0,)), ((), ())),
        preferred_element_type=jnp.float32,
    )

    slot = jax.lax.broadcasted_iota(jnp.int32, (NSQ, 1), 0).astype(jnp.float32)
    sel = (pos == slot).astype(jnp.float32) * keep
    fi = jax.lax.broadcasted_iota(jnp.int32, (NSQ, 1), 0)
    r_col = (fi // N).astype(jnp.float32)
    c_col_idx = (fi % N).astype(jnp.float32)
    rows_out = jax.lax.dot_general(
        sel, r_col, (((1,), (0,)), ((), ())),
        preferred_element_type=jnp.float32,
    )
    cols_out = jax.lax.dot_general(
        sel, c_col_idx, (((1,), (0,)), ((), ())),
        preferred_element_type=jnp.float32,
    )
    rows_ref[...] = rows_out.astype(jnp.int32)
    cols_ref[...] = cols_out.astype(jnp.int32)


def kernel(H):
    X = H.reshape(B, W, N, TW)
    x_nodes, csum = pl.pallas_call(
        _copy_corr_kernel3,
        grid=(NSTEPS,),
        in_specs=[
            pl.BlockSpec(memory_space=pltpu.MemorySpace.HBM),
        ],
        out_specs=[
            pl.BlockSpec(memory_space=pltpu.MemorySpace.HBM),
            pl.BlockSpec((N, N), lambda i: (0, 0)),
        ],
        out_shape=[
            jax.ShapeDtypeStruct((B, N, W * TW), jnp.float32),
            jax.ShapeDtypeStruct((N, N), jnp.float32),
        ],
        scratch_shapes=[
            pltpu.VMEM((RING, N, TW), jnp.float32),
            pltpu.SemaphoreType.DMA((RING,)),
            pltpu.SemaphoreType.DMA((RING,)),
        ],
        compiler_params=pltpu.CompilerParams(
            dimension_semantics=("arbitrary",),
        ),
    )(X)

    c_row = csum.reshape(1, NSQ)
    c_col = csum.reshape(NSQ, 1)
    rows, cols = pl.pallas_call(
        _edge_kernel,
        in_specs=[
            pl.BlockSpec((1, NSQ), lambda: (0, 0)),
            pl.BlockSpec((NSQ, 1), lambda: (0, 0)),
        ],
        out_specs=[
            pl.BlockSpec((NSQ, 1), lambda: (0, 0)),
            pl.BlockSpec((NSQ, 1), lambda: (0, 0)),
        ],
        out_shape=[
            jax.ShapeDtypeStruct((NSQ, 1), jnp.int32),
            jax.ShapeDtypeStruct((NSQ, 1), jnp.int32),
        ],
    )(c_row, c_col)

    edge_index = jnp.stack([rows[:NNZ, 0], cols[:NNZ, 0]], axis=0)
    return (x_nodes, edge_index)
